# baseline (speedup 1.0000x reference)
import functools
import jax
import jax.numpy as jnp
from jax import lax
from jax.experimental import pallas as pl
from jax.experimental.pallas import tpu as pltpu
from jax.experimental.pallas import tpu_sc as plsc

N_NODES = 10000
D = 256
HD = 128
NC = 2
NS = 16
CHUNK = 128
N_PAD = 10112
ZR = N_PAD // NS
GRP = 2
SG = 16
BN = 1000


def _make_seg_sum(cpt):
  mesh = plsc.VectorSubcoreMesh(
      core_axis_name="c", subcore_axis_name="s", num_cores=NC, num_subcores=NS)

  @functools.partial(
      pl.kernel,
      out_type=jax.ShapeDtypeStruct((NC, N_PAD, HD), jnp.float32),
      mesh=mesh,
      scratch_types=[
          pltpu.VMEM((SG, CHUNK), jnp.int32),
          pltpu.VMEM((SG, CHUNK), jnp.int32),
      ] + [pltpu.VMEM((CHUNK, HD), jnp.float32)] * GRP + [
          pltpu.VMEM_SHARED((N_PAD, HD), jnp.float32),
          pltpu.SemaphoreType.DMA,
          pltpu.SemaphoreType.DMA,
      ],
  )
  def seg_sum(p_flat, gsrc, gdst, zrow, out, src_v, dst_v, b0, b1,
              acc, sem_g, sem_s):
    bufs = [b0, b1]
    c = lax.axis_index("c")
    s = lax.axis_index("s")
    pltpu.sync_copy(zrow, acc.at[pl.ds(s * ZR, ZR)])
    plsc.subcore_barrier()

    def supergroup(t, carry):
      pltpu.sync_copy(gsrc.at[c, s, pl.ds(t * SG, SG)], src_v)
      pltpu.sync_copy(gdst.at[s, pl.ds(t * SG, SG)], dst_v)
      pltpu.async_copy(p_flat.at[src_v.at[0]], bufs[0], sem_g)
      scat = {}
      for j in range(SG):
        pltpu.make_async_copy(
            p_flat.at[src_v.at[j]], bufs[j % 2], sem_g).wait()
        scat[j] = pltpu.async_copy(
            bufs[j % 2], acc.at[dst_v.at[j]], sem_s, add=True)
        if j > 0:
          scat[j - 1].wait()
        if j + 1 < SG:
          pltpu.async_copy(p_flat.at[src_v.at[j + 1]], bufs[(j + 1) % 2],
                           sem_g)
      scat[SG - 1].wait()
      return carry

    lax.fori_loop(0, cpt // SG, supergroup, 0)
    plsc.subcore_barrier()
    pltpu.sync_copy(acc.at[pl.ds(s * ZR, ZR)],
                    out.at[c, pl.ds(s * ZR, ZR)])

  return seg_sum


def _make_deg(cpt):
  mesh = plsc.VectorSubcoreMesh(
      core_axis_name="c", subcore_axis_name="s", num_cores=NC, num_subcores=NS)

  @functools.partial(
      pl.kernel,
      out_type=jax.ShapeDtypeStruct((NC, N_PAD, HD), jnp.float32),
      mesh=mesh,
      scratch_types=[
          pltpu.VMEM((cpt, CHUNK), jnp.int32),
          pltpu.VMEM((CHUNK, HD), jnp.float32),
          pltpu.VMEM_SHARED((N_PAD, HD), jnp.float32),
          pltpu.SemaphoreType.DMA,
      ],
  )
  def deg(didx, ones, z16, out, idx_v, ones_v, acc, sem_s):
    c = lax.axis_index("c")
    s = lax.axis_index("s")
    pltpu.sync_copy(didx.at[c, s], idx_v)
    pltpu.sync_copy(ones, ones_v)
    pltpu.sync_copy(z16, acc.at[pl.ds(s * ZR, ZR)])
    plsc.subcore_barrier()

    def step(j, carry):
      pltpu.sync_copy(ones_v, acc.at[idx_v.at[j]], add=True)
      return carry

    lax.fori_loop(0, cpt, step, 0)
    plsc.subcore_barrier()
    pltpu.sync_copy(acc.at[pl.ds(s * ZR, ZR)],
                    out.at[c, pl.ds(s * ZR, ZR)])

  return deg


def _pre0_body(x_ref, w_ref, deg_ref, p_ref):
  ns = lax.rsqrt(jnp.maximum(deg_ref[0, :, 0:1], 1.0))
  p = jnp.dot(x_ref[...], w_ref[...],
              preferred_element_type=jnp.float32) * ns
  p_ref[0] = p[:, :HD]
  p_ref[1] = p[:, HD:]


def _mid_body(use_max, a_ref, deg_ref, b_ref, w_ref, m_ref, p_ref, m_out_ref):
  ns = lax.rsqrt(jnp.maximum(deg_ref[0, :, 0:1], 1.0))
  nd = lax.rsqrt(jnp.maximum(deg_ref[1, :, 0:1], 1.0))
  agg = jnp.concatenate([a_ref[0], a_ref[1]], axis=1)
  h = jnp.maximum(agg * nd + b_ref[0], 0.0)
  m_new = jnp.maximum(m_ref[...], h)
  m_out_ref[...] = m_new
  x = m_new if use_max else h
  p = jnp.dot(x, w_ref[...], preferred_element_type=jnp.float32) * ns
  p_ref[0] = p[:, :HD]
  p_ref[1] = p[:, HD:]


def _final_body(a_ref, deg_ref, b_ref, o_ref):
  nd = lax.rsqrt(jnp.maximum(deg_ref[1, :, 0:1], 1.0))
  x = jnp.concatenate([a_ref[0], a_ref[1]], axis=1) * nd + b_ref[0]
  m = jnp.max(x, axis=1, keepdims=True)
  e = jnp.exp(x - m)
  lse = jnp.log(jnp.sum(e, axis=1, keepdims=True))
  o_ref[...] = x - m - lse


_GRID = (N_NODES // BN,)
_spec_x = pl.BlockSpec((BN, D), lambda i: (i, 0))
_spec_w = pl.BlockSpec((D, D), lambda i: (0, 0))
_spec_deg = pl.BlockSpec((2, BN, HD), lambda i: (0, i, 0))
_spec_b = pl.BlockSpec((1, D), lambda i: (0, 0))
_spec_p = pl.BlockSpec((2, BN, HD), lambda i: (0, i, 0))
_spec_m = pl.BlockSpec((BN, D), lambda i: (i, 0))


def _pre0(x, w, deg):
  return pl.pallas_call(
      _pre0_body,
      grid=_GRID,
      in_specs=[_spec_x, _spec_w, _spec_deg],
      out_specs=_spec_p,
      out_shape=jax.ShapeDtypeStruct((2, N_NODES, HD), jnp.float32),
  )(x, w, deg)


def _mid(a, deg, b, w, m, use_max):
  return pl.pallas_call(
      functools.partial(_mid_body, use_max),
      grid=_GRID,
      in_specs=[_spec_p, _spec_deg, _spec_b, _spec_w, _spec_m],
      out_specs=[_spec_p, _spec_m],
      out_shape=[
          jax.ShapeDtypeStruct((2, N_NODES, HD), jnp.float32),
          jax.ShapeDtypeStruct((N_NODES, D), jnp.float32),
      ],
  )(a, deg, b, w, m)


def _final(a, deg, b):
  return pl.pallas_call(
      _final_body,
      grid=_GRID,
      in_specs=[_spec_p, _spec_deg, _spec_b],
      out_specs=pl.BlockSpec((BN, D), lambda i: (i, 0)),
      out_shape=jax.ShapeDtypeStruct((N_NODES, D), jnp.float32),
  )(a, deg, b)


def kernel(features, edge_index, W0, b0, W1, b1, W2, b2, W3, b3, W4, b4,
           W5, b5, W6, b6):
  src = edge_index[0].astype(jnp.int32)
  dst = edge_index[1].astype(jnp.int32)
  e = src.shape[0]
  per_tile = NS * CHUNK
  cpt = SG * (-(-e // (per_tile * SG)))
  e_pad = cpt * per_tile
  pad = e_pad - e

  src_g = jnp.concatenate([src, jnp.zeros((pad,), jnp.int32)])
  dst_g = jnp.concatenate([dst, jnp.full((pad,), N_NODES, jnp.int32)])
  src_d = jnp.concatenate([src, jnp.full((pad,), N_NODES, jnp.int32)])
  gsrc = jnp.stack([src_g, src_g + N_NODES]).reshape(NC, NS, cpt, CHUNK)
  gdst = dst_g.reshape(NS, cpt, CHUNK)
  didx = jnp.stack([src_d, dst_g]).reshape(NC, NS, cpt, CHUNK)

  zrow = jnp.zeros((ZR, HD), jnp.float32)
  z16 = jnp.zeros((ZR, HD), jnp.float32)
  ones = jnp.ones((CHUNK, HD), jnp.float32)

  seg_sum = _make_seg_sum(cpt)
  deg_k = _make_deg(cpt)

  deg = deg_k(didx, ones, z16)

  bs = [x.reshape(1, D) for x in (b0, b1, b2, b3, b4, b5, b6)]
  ws = [W1, W2, W3, W4, W5, W6]

  p = _pre0(features, W0, deg)
  m = jnp.zeros((N_NODES, D), jnp.float32)
  for i in range(6):
    a = seg_sum(p.reshape(NC * N_NODES, HD), gsrc, gdst, zrow)
    p, m = _mid(a, deg, bs[i], ws[i], m, use_max=(i == 5))
  a = seg_sum(p.reshape(NC * N_NODES, HD), gsrc, gdst, zrow)
  return _final(a, deg, bs[6])


# speedup vs baseline: 1.2539x; 1.2539x over previous
import functools
import jax
import jax.numpy as jnp
from jax import lax
from jax.experimental import pallas as pl
from jax.experimental.pallas import tpu as pltpu
from jax.experimental.pallas import tpu_sc as plsc

N_NODES = 10000
D = 256
HD = 128
NC = 2
NS = 16
CHUNK = 128
N_PAD = 10112
ZR = N_PAD // NS
GRP = 2
BN = 1000


def _make_seg_sum(cpt):
  mesh = plsc.VectorSubcoreMesh(
      core_axis_name="c", subcore_axis_name="s", num_cores=NC, num_subcores=NS)

  @functools.partial(
      pl.kernel,
      out_type=jax.ShapeDtypeStruct((NC, N_PAD, HD), jnp.float32),
      mesh=mesh,
      scratch_types=[
          pltpu.VMEM((cpt, CHUNK), jnp.int32),
          pltpu.VMEM((cpt, CHUNK), jnp.int32),
          pltpu.VMEM((CHUNK, HD), jnp.float32),
          pltpu.VMEM_SHARED((N_PAD, HD), jnp.float32),
          pltpu.SemaphoreType.DMA,
      ],
  )
  def seg_sum(p_flat, gsrc, gdst, zrow, out, src_v, dst_v, buf, acc, sem_g):
    c = lax.axis_index("c")
    s = lax.axis_index("s")
    pltpu.sync_copy(gsrc.at[c, s], src_v)
    pltpu.sync_copy(gdst.at[s], dst_v)
    pltpu.sync_copy(zrow, acc.at[pl.ds(s * ZR, ZR)])
    plsc.subcore_barrier()

    def step(j, carry):
      pltpu.async_copy(p_flat.at[src_v.at[j]], buf, sem_g).wait()
      pltpu.sync_copy(buf, acc.at[dst_v.at[j]], add=True)
      return carry

    lax.fori_loop(0, cpt, step, 0)
    plsc.subcore_barrier()
    pltpu.sync_copy(acc.at[pl.ds(s * ZR, ZR)],
                    out.at[c, pl.ds(s * ZR, ZR)])

  return seg_sum


def _make_deg(cpt):
  mesh = plsc.VectorSubcoreMesh(
      core_axis_name="c", subcore_axis_name="s", num_cores=NC, num_subcores=NS)

  @functools.partial(
      pl.kernel,
      out_type=jax.ShapeDtypeStruct((NC, N_PAD, HD), jnp.float32),
      mesh=mesh,
      scratch_types=[
          pltpu.VMEM((cpt, CHUNK), jnp.int32),
          pltpu.VMEM((CHUNK, HD), jnp.float32),
          pltpu.VMEM_SHARED((N_PAD, HD), jnp.float32),
          pltpu.SemaphoreType.DMA,
      ],
  )
  def deg(didx, ones, z16, out, idx_v, ones_v, acc, sem_s):
    c = lax.axis_index("c")
    s = lax.axis_index("s")
    pltpu.sync_copy(didx.at[c, s], idx_v)
    pltpu.sync_copy(ones, ones_v)
    pltpu.sync_copy(z16, acc.at[pl.ds(s * ZR, ZR)])
    plsc.subcore_barrier()

    def step(j, carry):
      pltpu.sync_copy(ones_v, acc.at[idx_v.at[j]], add=True)
      return carry

    lax.fori_loop(0, cpt, step, 0)
    plsc.subcore_barrier()
    pltpu.sync_copy(acc.at[pl.ds(s * ZR, ZR)],
                    out.at[c, pl.ds(s * ZR, ZR)])

  return deg


def _pre0_body(x_ref, w_ref, deg_ref, p_ref):
  ns = lax.rsqrt(jnp.maximum(deg_ref[0, :, 0:1], 1.0))
  p = jnp.dot(x_ref[...], w_ref[...],
              preferred_element_type=jnp.float32) * ns
  p_ref[0] = p[:, :HD]
  p_ref[1] = p[:, HD:]


def _mid_body(use_max, a_ref, deg_ref, b_ref, w_ref, m_ref, p_ref, m_out_ref):
  ns = lax.rsqrt(jnp.maximum(deg_ref[0, :, 0:1], 1.0))
  nd = lax.rsqrt(jnp.maximum(deg_ref[1, :, 0:1], 1.0))
  agg = jnp.concatenate([a_ref[0], a_ref[1]], axis=1)
  h = jnp.maximum(agg * nd + b_ref[0], 0.0)
  m_new = jnp.maximum(m_ref[...], h)
  m_out_ref[...] = m_new
  x = m_new if use_max else h
  p = jnp.dot(x, w_ref[...], preferred_element_type=jnp.float32) * ns
  p_ref[0] = p[:, :HD]
  p_ref[1] = p[:, HD:]


def _final_body(a_ref, deg_ref, b_ref, o_ref):
  nd = lax.rsqrt(jnp.maximum(deg_ref[1, :, 0:1], 1.0))
  x = jnp.concatenate([a_ref[0], a_ref[1]], axis=1) * nd + b_ref[0]
  m = jnp.max(x, axis=1, keepdims=True)
  e = jnp.exp(x - m)
  lse = jnp.log(jnp.sum(e, axis=1, keepdims=True))
  o_ref[...] = x - m - lse


_GRID = (N_NODES // BN,)
_spec_x = pl.BlockSpec((BN, D), lambda i: (i, 0))
_spec_w = pl.BlockSpec((D, D), lambda i: (0, 0))
_spec_deg = pl.BlockSpec((2, BN, HD), lambda i: (0, i, 0))
_spec_b = pl.BlockSpec((1, D), lambda i: (0, 0))
_spec_p = pl.BlockSpec((2, BN, HD), lambda i: (0, i, 0))
_spec_m = pl.BlockSpec((BN, D), lambda i: (i, 0))


def _pre0(x, w, deg):
  return pl.pallas_call(
      _pre0_body,
      grid=_GRID,
      in_specs=[_spec_x, _spec_w, _spec_deg],
      out_specs=_spec_p,
      out_shape=jax.ShapeDtypeStruct((2, N_NODES, HD), jnp.float32),
  )(x, w, deg)


def _mid(a, deg, b, w, m, use_max):
  return pl.pallas_call(
      functools.partial(_mid_body, use_max),
      grid=_GRID,
      in_specs=[_spec_p, _spec_deg, _spec_b, _spec_w, _spec_m],
      out_specs=[_spec_p, _spec_m],
      out_shape=[
          jax.ShapeDtypeStruct((2, N_NODES, HD), jnp.float32),
          jax.ShapeDtypeStruct((N_NODES, D), jnp.float32),
      ],
  )(a, deg, b, w, m)


def _final(a, deg, b):
  return pl.pallas_call(
      _final_body,
      grid=_GRID,
      in_specs=[_spec_p, _spec_deg, _spec_b],
      out_specs=pl.BlockSpec((BN, D), lambda i: (i, 0)),
      out_shape=jax.ShapeDtypeStruct((N_NODES, D), jnp.float32),
  )(a, deg, b)


def kernel(features, edge_index, W0, b0, W1, b1, W2, b2, W3, b3, W4, b4,
           W5, b5, W6, b6):
  src = edge_index[0].astype(jnp.int32)
  dst = edge_index[1].astype(jnp.int32)
  e = src.shape[0]
  per_tile = NS * CHUNK
  cpt = -(-e // per_tile)
  e_pad = cpt * per_tile
  pad = e_pad - e

  src_g = jnp.concatenate([src, jnp.zeros((pad,), jnp.int32)])
  dst_g = jnp.concatenate([dst, jnp.full((pad,), N_NODES, jnp.int32)])
  src_d = jnp.concatenate([src, jnp.full((pad,), N_NODES, jnp.int32)])
  gsrc = jnp.stack([src_g, src_g + N_NODES]).reshape(NC, NS, cpt, CHUNK)
  gdst = dst_g.reshape(NS, cpt, CHUNK)
  didx = jnp.stack([src_d, dst_g]).reshape(NC, NS, cpt, CHUNK)

  zrow = jnp.zeros((ZR, HD), jnp.float32)
  z16 = jnp.zeros((ZR, HD), jnp.float32)
  ones = jnp.ones((CHUNK, HD), jnp.float32)

  seg_sum = _make_seg_sum(cpt)
  deg_k = _make_deg(cpt)

  deg = deg_k(didx, ones, z16)

  bs = [x.reshape(1, D) for x in (b0, b1, b2, b3, b4, b5, b6)]
  ws = [W1, W2, W3, W4, W5, W6]

  p = _pre0(features, W0, deg)
  m = jnp.zeros((N_NODES, D), jnp.float32)
  for i in range(6):
    a = seg_sum(p.reshape(NC * N_NODES, HD), gsrc, gdst, zrow)
    p, m = _mid(a, deg, bs[i], ws[i], m, use_max=(i == 5))
  a = seg_sum(p.reshape(NC * N_NODES, HD), gsrc, gdst, zrow)
  return _final(a, deg, bs[6])
